# baseline (speedup 1.0000x reference)
import functools

import jax
import jax.numpy as jnp
import numpy as np
from jax import lax
from jax.experimental import pallas as pl
from jax.experimental.pallas import tpu as pltpu
from jax.experimental.pallas import tpu_sc as plsc

N = 50000
E = 1600000
EA = 800000
P = 1600000

NPAD = 50048
EPAD = 1638400
NBLK = 100
LAST_BLK = 97
LAST_LEN = E - LAST_BLK * 16384
CH = 2000
NC, NS = 2, 16

R_HALF = N * 4 // 2
R_DUMP = 16
A_CHUNK_ATOMS = 12512
A_ROWS = A_CHUNK_ATOMS * 10
A_LAST_ROWS = 10 * N - 3 * A_ROWS
A_DUMP = 64
SCH = 800
NZ = 6336
NZA = 7872

_mesh = plsc.VectorSubcoreMesh(core_axis_name="c", subcore_axis_name="s")


def _iota16():
    return lax.iota(jnp.int32, 16)


def _g12_body(species_hbm, esrc_hbm, edst_hbm, aedst_hbm, angd_hbm, angsw_hbm,
              rdist_hbm, rsw_hbm, ang_hbm,
              ridx_out, packed_out, rdp_out, rswp_out, angp_out,
              table_v, a_v, b_v, r_v, pk_v, in_v, out_v):
    c = lax.axis_index("c")
    s = lax.axis_index("s")
    wid = c * NS + s

    pltpu.sync_copy(species_hbm, table_v)

    def permute(src_hbm, dst_out):
        def blkfn(t, _):
            blk = wid + 32 * t

            @pl.when(blk < LAST_BLK)
            def _():
                pltpu.sync_copy(src_hbm.at[pl.ds(blk * 16384, 16384)], in_v)

            @pl.when(blk == LAST_BLK)
            def _():
                pltpu.sync_copy(src_hbm.at[pl.ds(blk * 16384, LAST_LEN)],
                                in_v.at[pl.ds(0, LAST_LEN)])

            @pl.when(blk <= LAST_BLK)
            def _():
                def grp(j, _):
                    vals = in_v[pl.ds(j * 16, 16)]
                    it = _iota16()
                    rr = (2 * j + (it >> 3)) & 127
                    q = 128 * rr + 8 * (j >> 6) + (it & 7)
                    plsc.store_scatter(out_v, [q], vals)
                    return 0

                lax.fori_loop(0, 1024, grp, 0)
                pltpu.sync_copy(out_v, dst_out.at[pl.ds(blk * 16384, 16384)])
            return 0

        lax.fori_loop(0, 4, blkfn, 0)

    permute(rdist_hbm, rdp_out)
    permute(rsw_hbm, rswp_out)
    permute(ang_hbm, angp_out)

    def g1_chunk(t, _):
        chunk = wid + 32 * t
        base = chunk * CH
        pltpu.sync_copy(esrc_hbm.at[pl.ds(base, CH)], a_v)
        pltpu.sync_copy(edst_hbm.at[pl.ds(base, CH)], b_v)

        def vec(j, _):
            src16 = a_v[pl.ds(j * 16, 16)]
            dst16 = b_v[pl.ds(j * 16, 16)]
            sp = plsc.load_gather(table_v, [dst16])
            idx4 = jnp.maximum(sp - 5, 0)
            r_v[pl.ds(j * 16, 16)] = src16 * 4 + idx4
            return 0

        lax.fori_loop(0, CH // 16, vec, 0)
        pltpu.sync_copy(r_v, ridx_out.at[pl.ds(base, CH)])
        return 0

    lax.fori_loop(0, E // CH // 32, g1_chunk, 0)

    def g2_chunk(t, _):
        chunk = s + NS * t
        base = chunk * CH
        pltpu.sync_copy(aedst_hbm.at[pl.ds(base, CH)], a_v)
        pltpu.sync_copy(angd_hbm.at[pl.ds(base, CH)], b_v)
        pltpu.sync_copy(angsw_hbm.at[pl.ds(base, CH)], r_v)

        def vec(j, _):
            ae16 = a_v[pl.ds(j * 16, 16)]
            d16 = plsc.bitcast(b_v[pl.ds(j * 16, 16)], jnp.float32)
            sw16 = r_v[pl.ds(j * 16, 16)]
            sp = plsc.load_gather(table_v, [ae16])
            idx4 = jnp.maximum(sp - 5, 0)
            psw = (sw16 & jnp.int32(-4)) | idx4
            rows = _iota16() + j * 16
            plsc.store_scatter(pk_v, [rows, jnp.zeros((16,), jnp.int32)], d16)
            plsc.store_scatter(pk_v, [rows, jnp.ones((16,), jnp.int32)],
                               plsc.bitcast(psw, jnp.float32))
            return 0

        lax.fori_loop(0, CH // 16, vec, 0)
        pltpu.sync_copy(pk_v, packed_out.at[pl.ds(c * EA + base, CH)])
        return 0

    lax.fori_loop(0, EA // CH // NS, g2_chunk, 0)


def _g12(species_pad, edge_src, edge_dst, ang_edge_dst, ang_d_bits, ang_sw_bits, rad_d, rad_sw, ang):
    f = pl.kernel(
        _g12_body,
        out_type=[
            jax.ShapeDtypeStruct((E,), jnp.int32),
            jax.ShapeDtypeStruct((2 * EA, 8), jnp.float32),
            jax.ShapeDtypeStruct((EPAD,), jnp.float32),
            jax.ShapeDtypeStruct((EPAD,), jnp.float32),
            jax.ShapeDtypeStruct((EPAD,), jnp.float32),
        ],
        mesh=_mesh,
        compiler_params=pltpu.CompilerParams(needs_layout_passes=False, use_tc_tiling_on_sc=False),
        scratch_types=[
            pltpu.VMEM((NPAD,), jnp.int32),
            pltpu.VMEM((CH,), jnp.int32),
            pltpu.VMEM((CH,), jnp.int32),
            pltpu.VMEM((CH,), jnp.int32),
            pltpu.VMEM((CH, 8), jnp.float32),
            pltpu.VMEM((16384,), jnp.float32),
            pltpu.VMEM((16384,), jnp.float32),
        ],
    )
    return f(species_pad, edge_src, edge_dst, ang_edge_dst, ang_d_bits,
             ang_sw_bits, rad_d, rad_sw, ang)


def _g3_body(packed_hbm, asrc_hbm, adst_hbm, cen_hbm,
             d12_out, swp_out, aidx_out,
             sidx_v, didx_v, cen_v, rows_s, rows_d, o1_v, o2_v, o3_v):
    c = lax.axis_index("c")
    s = lax.axis_index("s")
    wid = c * NS + s
    off = c * EA

    def chunkfn(t, _):
        chunk = wid + 32 * t
        base = chunk * CH
        pltpu.sync_copy(asrc_hbm.at[pl.ds(base, CH)], sidx_v)
        pltpu.sync_copy(adst_hbm.at[pl.ds(base, CH)], didx_v)
        pltpu.sync_copy(cen_hbm.at[pl.ds(base, CH)], cen_v)

        def addoff(j, _):
            sidx_v[pl.ds(j * 16, 16)] = sidx_v[pl.ds(j * 16, 16)] + off
            didx_v[pl.ds(j * 16, 16)] = didx_v[pl.ds(j * 16, 16)] + off
            return 0

        lax.fori_loop(0, CH // 16, addoff, 0)
        pltpu.sync_copy(packed_hbm.at[sidx_v], rows_s)
        pltpu.sync_copy(packed_hbm.at[didx_v], rows_d)

        def vec(j, _):
            rows = _iota16() + j * 16
            zero = jnp.zeros((16,), jnp.int32)
            one = jnp.ones((16,), jnp.int32)
            d1 = plsc.load_gather(rows_s, [rows, zero])
            p1 = plsc.bitcast(plsc.load_gather(rows_s, [rows, one]), jnp.int32)
            d2 = plsc.load_gather(rows_d, [rows, zero])
            p2 = plsc.bitcast(plsc.load_gather(rows_d, [rows, one]), jnp.int32)
            t1 = p1 & 3
            t2 = p2 & 3
            sw1 = plsc.bitcast(p1 & jnp.int32(-4), jnp.float32)
            sw2 = plsc.bitcast(p2 & jnp.int32(-4), jnp.float32)
            o1_v[pl.ds(j * 16, 16)] = 0.5 * (d1 + d2)
            o2_v[pl.ds(j * 16, 16)] = 2.0 * sw1 * sw2
            i = jnp.minimum(t1, t2)
            jj = jnp.maximum(t1, t2)
            tri = 4 * i - (i * (i - 1)) // 2 + (jj - i)
            cen16 = cen_v[pl.ds(j * 16, 16)]
            o3_v[pl.ds(j * 16, 16)] = cen16 * 10 + tri
            return 0

        lax.fori_loop(0, CH // 16, vec, 0)
        pltpu.sync_copy(o1_v, d12_out.at[pl.ds(base, CH)])
        pltpu.sync_copy(o2_v, swp_out.at[pl.ds(base, CH)])
        pltpu.sync_copy(o3_v, aidx_out.at[pl.ds(base, CH)])
        return 0

    lax.fori_loop(0, P // CH // 32, chunkfn, 0)


def _g3(packed, angle_src, angle_dst, central_atom):
    f = pl.kernel(
        _g3_body,
        out_type=[
            jax.ShapeDtypeStruct((P,), jnp.float32),
            jax.ShapeDtypeStruct((P,), jnp.float32),
            jax.ShapeDtypeStruct((P,), jnp.int32),
        ],
        mesh=_mesh,
        compiler_params=pltpu.CompilerParams(needs_layout_passes=False, use_tc_tiling_on_sc=False),
        scratch_types=[
            pltpu.VMEM((CH,), jnp.int32),
            pltpu.VMEM((CH,), jnp.int32),
            pltpu.VMEM((CH,), jnp.int32),
            pltpu.VMEM((CH, 8), jnp.float32),
            pltpu.VMEM((CH, 8), jnp.float32),
            pltpu.VMEM((CH,), jnp.float32),
            pltpu.VMEM((CH,), jnp.float32),
            pltpu.VMEM((CH,), jnp.int32),
        ],
    )
    return f(packed, angle_src, angle_dst, central_atom)


TC_BR = 128
TC_GRID = NBLK


def _perm_pad(x):
    return (jnp.pad(x, (0, EPAD - E)).reshape(NBLK, 16, 128, 8)
            .transpose(0, 2, 1, 3).reshape(NBLK, 128, 128))


def _make_expand_mat():
    c = np.arange(128)[:, None]
    j = np.arange(2048)[None, :]
    return jnp.asarray((c == 8 * (j >> 7) + ((j & 127) >> 4)).astype(np.float32))


def _expand(x, M, scr):
    hi = x.astype(jnp.bfloat16).astype(jnp.float32)
    lo = x - hi
    dims = (((1,), (0,)), ((), ()))
    w = (lax.dot_general(hi, M, dims, preferred_element_type=jnp.float32)
         + lax.dot_general(lo, M, dims, preferred_element_type=jnp.float32))
    for g in range(16):
        scr[pl.ds(128 * g, 128), :] = w[:, 128 * g:128 * (g + 1)]
    return scr[...]


def _radial_terms_body(d_ref, sw_ref, m_ref, out_ref, scr):
    M = m_ref[...]
    d_exp = _expand(d_ref[0], M, scr)
    lane = lax.broadcasted_iota(jnp.int32, (1, 128), 1)
    shift = 0.8 + 0.275 * (lane & 15).astype(jnp.float32)
    x = d_exp - shift
    e = 0.25 * jnp.exp(-16.0 * x * x)
    sw_exp = _expand(sw_ref[0], M, scr)
    out_ref[...] = e * sw_exp


def _radial_terms(d_p, sw_p, M):
    return pl.pallas_call(
        _radial_terms_body,
        grid=(TC_GRID,),
        in_specs=[
            pl.BlockSpec((1, TC_BR, 128), lambda i: (i, 0, 0)),
            pl.BlockSpec((1, TC_BR, 128), lambda i: (i, 0, 0)),
            pl.BlockSpec((128, 2048), lambda i: (0, 0)),
        ],
        out_specs=pl.BlockSpec((2048, 128), lambda i: (i, 0)),
        out_shape=jax.ShapeDtypeStruct((EPAD // 8, 128), jnp.float32),
        scratch_shapes=[pltpu.VMEM((2048, 128), jnp.float32)],
    )(d_p, sw_p, M)


def _angular_terms_body(a_ref, d_ref, w_ref, m_ref, out_ref, scr):
    M = m_ref[...]
    a = a_ref[0]
    ca_exp = _expand(jnp.cos(a), M, scr)
    sa_exp = _expand(jnp.sin(a), M, scr)
    lane = lax.broadcasted_iota(jnp.int32, (1, 128), 1)
    k = lane & 3
    m = (lane >> 2) & 3
    czv = np.cos(np.pi / 8 + np.arange(4) * np.pi / 4).astype(np.float32)
    szv = np.sin(np.pi / 8 + np.arange(4) * np.pi / 4).astype(np.float32)
    cz = jnp.where(k == 0, czv[0], jnp.where(k == 1, czv[1],
         jnp.where(k == 2, czv[2], czv[3])))
    sz = jnp.where(k == 0, szv[0], jnp.where(k == 1, szv[1],
         jnp.where(k == 2, szv[2], szv[3])))
    f1 = 0.5 + 0.5 * (ca_exp * cz + sa_exp * sz)
    f1 = f1 * f1
    f1 = f1 * f1
    f1 = f1 * f1
    f1 = f1 * f1
    f1 = f1 * f1
    d_exp = _expand(d_ref[0], M, scr)
    sha = 0.8 + 0.675 * m.astype(jnp.float32)
    x = d_exp - sha
    f2 = jnp.exp(-8.0 * x * x)
    w_exp = _expand(w_ref[0], M, scr)
    out_ref[...] = (f1 * f2 * w_exp).astype(jnp.bfloat16)


def _angular_terms(a_p, d_p, w_p, M):
    return pl.pallas_call(
        _angular_terms_body,
        grid=(TC_GRID,),
        in_specs=[
            pl.BlockSpec((1, TC_BR, 128), lambda i: (i, 0, 0)),
            pl.BlockSpec((1, TC_BR, 128), lambda i: (i, 0, 0)),
            pl.BlockSpec((1, TC_BR, 128), lambda i: (i, 0, 0)),
            pl.BlockSpec((128, 2048), lambda i: (0, 0)),
        ],
        out_specs=pl.BlockSpec((2048, 128), lambda i: (i, 0)),
        out_shape=jax.ShapeDtypeStruct((EPAD // 8, 128), jnp.bfloat16),
        scratch_shapes=[pltpu.VMEM((2048, 128), jnp.float32)],
    )(a_p, d_p, w_p, M)


def _scatter_body(passes, idx_hbm, terms_hbm, zeros_hbm, out_ref,
                  acc, idx2_v, terms2_v, lidx_v, si0, si1, st0, st1,
                  *, acc_rows, dump, n_chunks):
    c = lax.axis_index("c")
    s = lax.axis_index("s")
    sem_i = (si0, si1)
    sem_t = (st0, st1)

    z_per_tile = (acc_rows + dump) // NS
    n_pairs = (n_chunks - 1) // 2
    assert n_chunks == 2 * n_pairs + 1

    def start_dma(t, b):
        base = (s + NS * t) * SCH
        pltpu.async_copy(idx_hbm.at[pl.ds(base, SCH)], idx2_v.at[b], sem_i[b])
        pltpu.async_copy(terms_hbm.at[pl.ds(base, SCH)], terms2_v.at[b], sem_t[b])

    for pidx, (lo_fn, rows0, rows1) in enumerate(passes):
        lo = lo_fn(c)
        arows = jnp.where(c == 0, rows0, rows1)
        if pidx > 0:
            plsc.subcore_barrier()

        pltpu.sync_copy(zeros_hbm.at[pl.ds(0, z_per_tile)],
                        acc.at[pl.ds(s * z_per_tile, z_per_tile)])
        plsc.subcore_barrier()

        def process(t, b):
            pltpu.make_async_copy(idx_hbm.at[pl.ds(0, SCH)], idx2_v.at[b],
                                  sem_i[b]).wait()
            pltpu.make_async_copy(terms_hbm.at[pl.ds(0, SCH)], terms2_v.at[b],
                                  sem_t[b]).wait()

            def vec(j, _):
                raw = idx2_v.at[b][pl.ds(j * 16, 16)]
                v = raw - lo
                ok = (v >= 0) & (v < arows)
                lidx_v[pl.ds(j * 16, 16)] = jnp.where(
                    ok, v, acc_rows + (raw & (dump - 1)))
                return 0

            lax.fori_loop(0, SCH // 16, vec, 0)
            pltpu.sync_copy(terms2_v.at[b], acc.at[lidx_v], add=True)

        start_dma(0, 0)

        def pairfn(i, _):
            start_dma(2 * i + 1, 1)
            process(2 * i, 0)
            start_dma(2 * i + 2, 0)
            process(2 * i + 1, 1)
            return 0

        lax.fori_loop(0, n_pairs, pairfn, 0)
        process(n_chunks - 1, 0)
        plsc.subcore_barrier()

        def emit_writeout(rows, cc):
            w_per_tile = rows // NS

            @pl.when(c == cc)
            def _():
                def wchunk(i, _):
                    r0 = s * w_per_tile + i * 1024
                    pltpu.sync_copy(acc.at[pl.ds(r0, 1024)],
                                    out_ref.at[pl.ds(lo + r0, 1024)])
                    return 0

                lax.fori_loop(0, w_per_tile // 1024, wchunk, 0)
                wrem = w_per_tile % 1024
                if wrem:
                    r0 = s * w_per_tile + (w_per_tile // 1024) * 1024
                    pltpu.sync_copy(acc.at[pl.ds(r0, wrem)],
                                    out_ref.at[pl.ds(lo + r0, wrem)])

        emit_writeout(rows0, 0)
        emit_writeout(rows1, 1)


def _seg_scatter(idx, terms, zeros, passes, acc_rows, dump, out_rows, n_chunks,
                 dtype=jnp.float32):
    body = functools.partial(_scatter_body, passes,
                             acc_rows=acc_rows, dump=dump, n_chunks=n_chunks)
    f = pl.kernel(
        body,
        out_type=jax.ShapeDtypeStruct((out_rows, 16), dtype),
        mesh=_mesh,
        compiler_params=pltpu.CompilerParams(needs_layout_passes=False, use_tc_tiling_on_sc=False),
        scratch_types=[
            pltpu.VMEM_SHARED((acc_rows + dump, 16), dtype),
            pltpu.VMEM((2, SCH), jnp.int32),
            pltpu.VMEM((2, SCH, 16), dtype),
            pltpu.VMEM((SCH,), jnp.int32),
            pltpu.SemaphoreType.DMA,
            pltpu.SemaphoreType.DMA,
            pltpu.SemaphoreType.DMA,
            pltpu.SemaphoreType.DMA,
        ],
    )
    return f(idx, terms, zeros)


def _radial_scatter(ridx, terms, zeros):
    return _seg_scatter(ridx, terms, zeros,
                        [(lambda c: c * R_HALF, R_HALF, R_HALF)],
                        R_HALF, R_DUMP, 4 * N, E // SCH // NS)


def _angular_scatter(aidx, terms, zeros):
    return _seg_scatter(aidx, terms, zeros,
                        [(lambda c: c * A_ROWS, A_ROWS, A_ROWS),
                         (lambda c: (c + 2) * A_ROWS, A_ROWS, A_LAST_ROWS)],
                        A_ROWS, A_DUMP, 10 * N, P // SCH // NS,
                        dtype=jnp.bfloat16)


def kernel(species, rad_distances, rad_switch, edge_src, edge_dst, angles,
           ang_distances, ang_switch, central_atom, angle_src, angle_dst,
           ang_edge_dst):
    species_pad = jnp.pad(species, (0, NPAD - N))
    ridx, packed, rdp, rswp, angp = _g12(
        species_pad, edge_src, edge_dst, ang_edge_dst,
        ang_distances.view(jnp.int32), ang_switch.view(jnp.int32),
        rad_distances, rad_switch, angles)
    d12, swp, aidx = _g3(packed, angle_src, angle_dst, central_atom)

    M = _make_expand_mat()
    rterms = _radial_terms(rdp.reshape(NBLK, 128, 128),
                           rswp.reshape(NBLK, 128, 128), M)
    aterms = _angular_terms(angp.reshape(NBLK, 128, 128),
                            _perm_pad(d12), _perm_pad(swp), M)

    zeros = jnp.zeros((NZ, 16), jnp.float32)
    zeros_bf = jnp.zeros((NZA, 16), jnp.bfloat16)
    raev = _radial_scatter(ridx, rterms.reshape(EPAD, 16), zeros).reshape(N, 64)
    aaev = _angular_scatter(aidx, aterms.reshape(EPAD, 16),
                            zeros_bf).astype(jnp.float32).reshape(N, 160)
    return jnp.concatenate((raev, aaev), axis=-1)


# speedup vs baseline: 1.2155x; 1.2155x over previous
import functools

import jax
import jax.numpy as jnp
import numpy as np
from jax import lax
from jax.experimental import pallas as pl
from jax.experimental.pallas import tpu as pltpu
from jax.experimental.pallas import tpu_sc as plsc

N = 50000
E = 1600000
EA = 800000
P = 1600000

NPAD = 50048
EPAD = 1638400
NBLK = 100
LAST_BLK = 97
LAST_LEN = E - LAST_BLK * 16384
CH = 2000
NC, NS = 2, 16

R_HALF = N * 4 // 2
R_DUMP = 16
A_CHUNK_ATOMS = 8336
A_ROWS = A_CHUNK_ATOMS * 10
A_LAST_ROWS = 10 * N - 5 * A_ROWS
A_DUMP = 64
SCH = 800
NZ = 6336

_mesh = plsc.VectorSubcoreMesh(core_axis_name="c", subcore_axis_name="s")


def _iota16():
    return lax.iota(jnp.int32, 16)


def _g12_body(species_hbm, esrc_hbm, edst_hbm, aedst_hbm, angd_hbm, angsw_hbm,
              rdist_hbm, rsw_hbm, ang_hbm,
              ridx_out, packed_out, rdp_out, rswp_out, angp_out,
              table_v, a_v, b_v, r_v, pk_v, in_v, out_v):
    c = lax.axis_index("c")
    s = lax.axis_index("s")
    wid = c * NS + s

    pltpu.sync_copy(species_hbm, table_v)

    def permute(src_hbm, dst_out):
        def blkfn(t, _):
            blk = wid + 32 * t

            @pl.when(blk < LAST_BLK)
            def _():
                pltpu.sync_copy(src_hbm.at[pl.ds(blk * 16384, 16384)], in_v)

            @pl.when(blk == LAST_BLK)
            def _():
                pltpu.sync_copy(src_hbm.at[pl.ds(blk * 16384, LAST_LEN)],
                                in_v.at[pl.ds(0, LAST_LEN)])

            @pl.when(blk <= LAST_BLK)
            def _():
                def grp(j, _):
                    vals = in_v[pl.ds(j * 16, 16)]
                    it = _iota16()
                    rr = (2 * j + (it >> 3)) & 127
                    q = 128 * rr + 8 * (j >> 6) + (it & 7)
                    plsc.store_scatter(out_v, [q], vals)
                    return 0

                lax.fori_loop(0, 1024, grp, 0)
                pltpu.sync_copy(out_v, dst_out.at[pl.ds(blk * 16384, 16384)])
            return 0

        lax.fori_loop(0, 4, blkfn, 0)

    permute(rdist_hbm, rdp_out)
    permute(rsw_hbm, rswp_out)
    permute(ang_hbm, angp_out)

    def g1_chunk(t, _):
        chunk = wid + 32 * t
        base = chunk * CH
        pltpu.sync_copy(esrc_hbm.at[pl.ds(base, CH)], a_v)
        pltpu.sync_copy(edst_hbm.at[pl.ds(base, CH)], b_v)

        def vec(j, _):
            src16 = a_v[pl.ds(j * 16, 16)]
            dst16 = b_v[pl.ds(j * 16, 16)]
            sp = plsc.load_gather(table_v, [dst16])
            idx4 = jnp.maximum(sp - 5, 0)
            r_v[pl.ds(j * 16, 16)] = src16 * 4 + idx4
            return 0

        lax.fori_loop(0, CH // 16, vec, 0)
        pltpu.sync_copy(r_v, ridx_out.at[pl.ds(base, CH)])
        return 0

    lax.fori_loop(0, E // CH // 32, g1_chunk, 0)

    def g2_chunk(t, _):
        chunk = s + NS * t
        base = chunk * CH
        pltpu.sync_copy(aedst_hbm.at[pl.ds(base, CH)], a_v)
        pltpu.sync_copy(angd_hbm.at[pl.ds(base, CH)], b_v)
        pltpu.sync_copy(angsw_hbm.at[pl.ds(base, CH)], r_v)

        def vec(j, _):
            ae16 = a_v[pl.ds(j * 16, 16)]
            d16 = plsc.bitcast(b_v[pl.ds(j * 16, 16)], jnp.float32)
            sw16 = r_v[pl.ds(j * 16, 16)]
            sp = plsc.load_gather(table_v, [ae16])
            idx4 = jnp.maximum(sp - 5, 0)
            psw = (sw16 & jnp.int32(-4)) | idx4
            rows = _iota16() + j * 16
            plsc.store_scatter(pk_v, [rows, jnp.zeros((16,), jnp.int32)], d16)
            plsc.store_scatter(pk_v, [rows, jnp.ones((16,), jnp.int32)],
                               plsc.bitcast(psw, jnp.float32))
            return 0

        lax.fori_loop(0, CH // 16, vec, 0)
        pltpu.sync_copy(pk_v, packed_out.at[pl.ds(c * EA + base, CH)])
        return 0

    lax.fori_loop(0, EA // CH // NS, g2_chunk, 0)


def _g12(species_pad, edge_src, edge_dst, ang_edge_dst, ang_d_bits, ang_sw_bits, rad_d, rad_sw, ang):
    f = pl.kernel(
        _g12_body,
        out_type=[
            jax.ShapeDtypeStruct((E,), jnp.int32),
            jax.ShapeDtypeStruct((2 * EA, 8), jnp.float32),
            jax.ShapeDtypeStruct((EPAD,), jnp.float32),
            jax.ShapeDtypeStruct((EPAD,), jnp.float32),
            jax.ShapeDtypeStruct((EPAD,), jnp.float32),
        ],
        mesh=_mesh,
        compiler_params=pltpu.CompilerParams(needs_layout_passes=False, use_tc_tiling_on_sc=False),
        scratch_types=[
            pltpu.VMEM((NPAD,), jnp.int32),
            pltpu.VMEM((CH,), jnp.int32),
            pltpu.VMEM((CH,), jnp.int32),
            pltpu.VMEM((CH,), jnp.int32),
            pltpu.VMEM((CH, 8), jnp.float32),
            pltpu.VMEM((16384,), jnp.float32),
            pltpu.VMEM((16384,), jnp.float32),
        ],
    )
    return f(species_pad, edge_src, edge_dst, ang_edge_dst, ang_d_bits,
             ang_sw_bits, rad_d, rad_sw, ang)


def _g3_body(packed_hbm, asrc_hbm, adst_hbm, cen_hbm,
             d12_out, swp_out, aidx_out,
             sidx_v, didx_v, cen_v, rows_s, rows_d, o1_v, o2_v, o3_v):
    c = lax.axis_index("c")
    s = lax.axis_index("s")
    wid = c * NS + s
    off = c * EA

    def chunkfn(t, _):
        chunk = wid + 32 * t
        base = chunk * CH
        pltpu.sync_copy(asrc_hbm.at[pl.ds(base, CH)], sidx_v)
        pltpu.sync_copy(adst_hbm.at[pl.ds(base, CH)], didx_v)
        pltpu.sync_copy(cen_hbm.at[pl.ds(base, CH)], cen_v)

        def addoff(j, _):
            sidx_v[pl.ds(j * 16, 16)] = sidx_v[pl.ds(j * 16, 16)] + off
            didx_v[pl.ds(j * 16, 16)] = didx_v[pl.ds(j * 16, 16)] + off
            return 0

        lax.fori_loop(0, CH // 16, addoff, 0)
        pltpu.sync_copy(packed_hbm.at[sidx_v], rows_s)
        pltpu.sync_copy(packed_hbm.at[didx_v], rows_d)

        def vec(j, _):
            rows = _iota16() + j * 16
            zero = jnp.zeros((16,), jnp.int32)
            one = jnp.ones((16,), jnp.int32)
            d1 = plsc.load_gather(rows_s, [rows, zero])
            p1 = plsc.bitcast(plsc.load_gather(rows_s, [rows, one]), jnp.int32)
            d2 = plsc.load_gather(rows_d, [rows, zero])
            p2 = plsc.bitcast(plsc.load_gather(rows_d, [rows, one]), jnp.int32)
            t1 = p1 & 3
            t2 = p2 & 3
            sw1 = plsc.bitcast(p1 & jnp.int32(-4), jnp.float32)
            sw2 = plsc.bitcast(p2 & jnp.int32(-4), jnp.float32)
            o1_v[pl.ds(j * 16, 16)] = 0.5 * (d1 + d2)
            o2_v[pl.ds(j * 16, 16)] = 2.0 * sw1 * sw2
            i = jnp.minimum(t1, t2)
            jj = jnp.maximum(t1, t2)
            tri = 4 * i - (i * (i - 1)) // 2 + (jj - i)
            cen16 = cen_v[pl.ds(j * 16, 16)]
            o3_v[pl.ds(j * 16, 16)] = cen16 * 10 + tri
            return 0

        lax.fori_loop(0, CH // 16, vec, 0)
        pltpu.sync_copy(o1_v, d12_out.at[pl.ds(base, CH)])
        pltpu.sync_copy(o2_v, swp_out.at[pl.ds(base, CH)])
        pltpu.sync_copy(o3_v, aidx_out.at[pl.ds(base, CH)])
        return 0

    lax.fori_loop(0, P // CH // 32, chunkfn, 0)


def _g3(packed, angle_src, angle_dst, central_atom):
    f = pl.kernel(
        _g3_body,
        out_type=[
            jax.ShapeDtypeStruct((P,), jnp.float32),
            jax.ShapeDtypeStruct((P,), jnp.float32),
            jax.ShapeDtypeStruct((P,), jnp.int32),
        ],
        mesh=_mesh,
        compiler_params=pltpu.CompilerParams(needs_layout_passes=False, use_tc_tiling_on_sc=False),
        scratch_types=[
            pltpu.VMEM((CH,), jnp.int32),
            pltpu.VMEM((CH,), jnp.int32),
            pltpu.VMEM((CH,), jnp.int32),
            pltpu.VMEM((CH, 8), jnp.float32),
            pltpu.VMEM((CH, 8), jnp.float32),
            pltpu.VMEM((CH,), jnp.float32),
            pltpu.VMEM((CH,), jnp.float32),
            pltpu.VMEM((CH,), jnp.int32),
        ],
    )
    return f(packed, angle_src, angle_dst, central_atom)


TC_BR = 128
TC_GRID = NBLK


def _perm_pad(x):
    return (jnp.pad(x, (0, EPAD - E)).reshape(NBLK, 16, 128, 8)
            .transpose(0, 2, 1, 3).reshape(NBLK, 128, 128))


def _make_expand_mat():
    c = np.arange(128)[:, None]
    j = np.arange(2048)[None, :]
    return jnp.asarray((c == 8 * (j >> 7) + ((j & 127) >> 4)).astype(np.float32))


def _expand(x, M, scr):
    hi = x.astype(jnp.bfloat16).astype(jnp.float32)
    lo = x - hi
    dims = (((1,), (0,)), ((), ()))
    w = (lax.dot_general(hi, M, dims, preferred_element_type=jnp.float32)
         + lax.dot_general(lo, M, dims, preferred_element_type=jnp.float32))
    for g in range(16):
        scr[pl.ds(128 * g, 128), :] = w[:, 128 * g:128 * (g + 1)]
    return scr[...]


def _radial_terms_body(d_ref, sw_ref, m_ref, out_ref, scr):
    M = m_ref[...]
    d_exp = _expand(d_ref[0], M, scr)
    lane = lax.broadcasted_iota(jnp.int32, (1, 128), 1)
    shift = 0.8 + 0.275 * (lane & 15).astype(jnp.float32)
    x = d_exp - shift
    e = 0.25 * jnp.exp(-16.0 * x * x)
    sw_exp = _expand(sw_ref[0], M, scr)
    out_ref[...] = e * sw_exp


def _radial_terms(d_p, sw_p, M):
    return pl.pallas_call(
        _radial_terms_body,
        grid=(TC_GRID,),
        in_specs=[
            pl.BlockSpec((1, TC_BR, 128), lambda i: (i, 0, 0)),
            pl.BlockSpec((1, TC_BR, 128), lambda i: (i, 0, 0)),
            pl.BlockSpec((128, 2048), lambda i: (0, 0)),
        ],
        out_specs=pl.BlockSpec((2048, 128), lambda i: (i, 0)),
        out_shape=jax.ShapeDtypeStruct((EPAD // 8, 128), jnp.float32),
        scratch_shapes=[pltpu.VMEM((2048, 128), jnp.float32)],
    )(d_p, sw_p, M)


def _angular_terms_body(a_ref, d_ref, w_ref, m_ref, out_ref, scr):
    M = m_ref[...]
    a = a_ref[0]
    ca_exp = _expand(jnp.cos(a), M, scr)
    sa_exp = _expand(jnp.sin(a), M, scr)
    lane = lax.broadcasted_iota(jnp.int32, (1, 128), 1)
    k = lane & 3
    m = (lane >> 2) & 3
    czv = np.cos(np.pi / 8 + np.arange(4) * np.pi / 4).astype(np.float32)
    szv = np.sin(np.pi / 8 + np.arange(4) * np.pi / 4).astype(np.float32)
    cz = jnp.where(k == 0, czv[0], jnp.where(k == 1, czv[1],
         jnp.where(k == 2, czv[2], czv[3])))
    sz = jnp.where(k == 0, szv[0], jnp.where(k == 1, szv[1],
         jnp.where(k == 2, szv[2], szv[3])))
    f1 = 0.5 + 0.5 * (ca_exp * cz + sa_exp * sz)
    f1 = f1 * f1
    f1 = f1 * f1
    f1 = f1 * f1
    f1 = f1 * f1
    f1 = f1 * f1
    d_exp = _expand(d_ref[0], M, scr)
    sha = 0.8 + 0.675 * m.astype(jnp.float32)
    x = d_exp - sha
    f2 = jnp.exp(-8.0 * x * x)
    w_exp = _expand(w_ref[0], M, scr)
    out_ref[...] = f1 * f2 * w_exp


def _angular_terms(a_p, d_p, w_p, M):
    return pl.pallas_call(
        _angular_terms_body,
        grid=(TC_GRID,),
        in_specs=[
            pl.BlockSpec((1, TC_BR, 128), lambda i: (i, 0, 0)),
            pl.BlockSpec((1, TC_BR, 128), lambda i: (i, 0, 0)),
            pl.BlockSpec((1, TC_BR, 128), lambda i: (i, 0, 0)),
            pl.BlockSpec((128, 2048), lambda i: (0, 0)),
        ],
        out_specs=pl.BlockSpec((2048, 128), lambda i: (i, 0)),
        out_shape=jax.ShapeDtypeStruct((EPAD // 8, 128), jnp.float32),
        scratch_shapes=[pltpu.VMEM((2048, 128), jnp.float32)],
    )(a_p, d_p, w_p, M)


def _scatter_body(passes, idx_hbm, terms_hbm, zeros_hbm, out_ref,
                  acc, idx2_v, terms2_v, lidx_v, si0, si1, st0, st1,
                  *, acc_rows, dump, n_chunks):
    c = lax.axis_index("c")
    s = lax.axis_index("s")
    sem_i = (si0, si1)
    sem_t = (st0, st1)

    z_per_tile = (acc_rows + dump) // NS
    n_pairs = (n_chunks - 1) // 2
    assert n_chunks == 2 * n_pairs + 1

    def start_dma(t, b):
        base = (s + NS * t) * SCH
        pltpu.async_copy(idx_hbm.at[pl.ds(base, SCH)], idx2_v.at[b], sem_i[b])
        pltpu.async_copy(terms_hbm.at[pl.ds(base, SCH)], terms2_v.at[b], sem_t[b])

    for pidx, (lo_fn, rows0, rows1) in enumerate(passes):
        lo = lo_fn(c)
        arows = jnp.where(c == 0, rows0, rows1)
        if pidx > 0:
            plsc.subcore_barrier()

        pltpu.sync_copy(zeros_hbm.at[pl.ds(0, z_per_tile)],
                        acc.at[pl.ds(s * z_per_tile, z_per_tile)])
        plsc.subcore_barrier()

        def process(t, b):
            pltpu.make_async_copy(idx_hbm.at[pl.ds(0, SCH)], idx2_v.at[b],
                                  sem_i[b]).wait()
            pltpu.make_async_copy(terms_hbm.at[pl.ds(0, SCH)], terms2_v.at[b],
                                  sem_t[b]).wait()

            def vec(j, _):
                raw = idx2_v.at[b][pl.ds(j * 16, 16)]
                v = raw - lo
                ok = (v >= 0) & (v < arows)
                lidx_v[pl.ds(j * 16, 16)] = jnp.where(
                    ok, v, acc_rows + (raw & (dump - 1)))
                return 0

            lax.fori_loop(0, SCH // 16, vec, 0)
            pltpu.sync_copy(terms2_v.at[b], acc.at[lidx_v], add=True)

        start_dma(0, 0)

        def pairfn(i, _):
            start_dma(2 * i + 1, 1)
            process(2 * i, 0)
            start_dma(2 * i + 2, 0)
            process(2 * i + 1, 1)
            return 0

        lax.fori_loop(0, n_pairs, pairfn, 0)
        process(n_chunks - 1, 0)
        plsc.subcore_barrier()

        def emit_writeout(rows, cc):
            w_per_tile = rows // NS

            @pl.when(c == cc)
            def _():
                def wchunk(i, _):
                    r0 = s * w_per_tile + i * 1024
                    pltpu.sync_copy(acc.at[pl.ds(r0, 1024)],
                                    out_ref.at[pl.ds(lo + r0, 1024)])
                    return 0

                lax.fori_loop(0, w_per_tile // 1024, wchunk, 0)
                wrem = w_per_tile % 1024
                if wrem:
                    r0 = s * w_per_tile + (w_per_tile // 1024) * 1024
                    pltpu.sync_copy(acc.at[pl.ds(r0, wrem)],
                                    out_ref.at[pl.ds(lo + r0, wrem)])

        emit_writeout(rows0, 0)
        emit_writeout(rows1, 1)


def _seg_scatter(idx, terms, zeros, passes, acc_rows, dump, out_rows, n_chunks,
                 dtype=jnp.float32):
    body = functools.partial(_scatter_body, passes,
                             acc_rows=acc_rows, dump=dump, n_chunks=n_chunks)
    f = pl.kernel(
        body,
        out_type=jax.ShapeDtypeStruct((out_rows, 16), dtype),
        mesh=_mesh,
        compiler_params=pltpu.CompilerParams(needs_layout_passes=False, use_tc_tiling_on_sc=False),
        scratch_types=[
            pltpu.VMEM_SHARED((acc_rows + dump, 16), dtype),
            pltpu.VMEM((2, SCH), jnp.int32),
            pltpu.VMEM((2, SCH, 16), dtype),
            pltpu.VMEM((SCH,), jnp.int32),
            pltpu.SemaphoreType.DMA,
            pltpu.SemaphoreType.DMA,
            pltpu.SemaphoreType.DMA,
            pltpu.SemaphoreType.DMA,
        ],
    )
    return f(idx, terms, zeros)


def _radial_scatter(ridx, terms, zeros):
    return _seg_scatter(ridx, terms, zeros,
                        [(lambda c: c * R_HALF, R_HALF, R_HALF)],
                        R_HALF, R_DUMP, 4 * N, E // SCH // NS)


def _angular_scatter(aidx, terms, zeros):
    return _seg_scatter(aidx, terms, zeros,
                        [(lambda c: c * A_ROWS, A_ROWS, A_ROWS),
                         (lambda c: (c + 2) * A_ROWS, A_ROWS, A_ROWS),
                         (lambda c: (c + 4) * A_ROWS, A_ROWS, A_LAST_ROWS)],
                        A_ROWS, A_DUMP, 10 * N, P // SCH // NS)


def kernel(species, rad_distances, rad_switch, edge_src, edge_dst, angles,
           ang_distances, ang_switch, central_atom, angle_src, angle_dst,
           ang_edge_dst):
    species_pad = jnp.pad(species, (0, NPAD - N))
    ridx, packed, rdp, rswp, angp = _g12(
        species_pad, edge_src, edge_dst, ang_edge_dst,
        ang_distances.view(jnp.int32), ang_switch.view(jnp.int32),
        rad_distances, rad_switch, angles)
    d12, swp, aidx = _g3(packed, angle_src, angle_dst, central_atom)

    M = _make_expand_mat()
    rterms = _radial_terms(rdp.reshape(NBLK, 128, 128),
                           rswp.reshape(NBLK, 128, 128), M)
    aterms = _angular_terms(angp.reshape(NBLK, 128, 128),
                            _perm_pad(d12), _perm_pad(swp), M)

    zeros = jnp.zeros((NZ, 16), jnp.float32)
    raev = _radial_scatter(ridx, rterms.reshape(EPAD, 16), zeros).reshape(N, 64)
    aaev = _angular_scatter(aidx, aterms.reshape(EPAD, 16), zeros).reshape(N, 160)
    return jnp.concatenate((raev, aaev), axis=-1)
